# baseline (speedup 1.0000x reference)
import jax
import jax.numpy as jnp
from jax.experimental import pallas as pl
from jax.experimental.pallas import tpu as pltpu

_NCHUNKS = 8


def _chunk_bounds(n):
    nchunks = min(_NCHUNKS, n)
    base = n // nchunks
    rem = n % nchunks
    bounds = []
    start = 0
    for i in range(nchunks):
        size = base + (1 if i < rem else 0)
        bounds.append((start, size))
        start += size
    return bounds


def _pad_body(x_ref, out_ref, mask_ref, sems):
    bounds = _chunk_bounds(x_ref.shape[0])
    for i, (start, size) in enumerate(bounds):
        pltpu.make_async_copy(
            x_ref.at[pl.ds(start, size)],
            out_ref.at[pl.ds(start, size)],
            sems.at[i],
        ).start()
    mask_ref[...] = jnp.ones(mask_ref.shape, dtype=jnp.bool_)
    for i, (start, size) in enumerate(bounds):
        pltpu.make_async_copy(
            x_ref.at[pl.ds(start, size)],
            out_ref.at[pl.ds(start, size)],
            sems.at[i],
        ).wait()


def kernel(bags):
    b, s, d = bags.shape
    padded, mask = pl.pallas_call(
        _pad_body,
        out_shape=(
            jax.ShapeDtypeStruct((b, s, d), bags.dtype),
            jax.ShapeDtypeStruct((b, s), jnp.bool_),
        ),
        in_specs=[pl.BlockSpec(memory_space=pl.ANY)],
        out_specs=(
            pl.BlockSpec(memory_space=pl.ANY),
            pl.BlockSpec(memory_space=pltpu.MemorySpace.VMEM),
        ),
        scratch_shapes=[pltpu.SemaphoreType.DMA((_NCHUNKS,))],
    )(bags)
    return (padded, mask)


# speedup vs baseline: 46.8173x; 46.8173x over previous
import jax
import jax.numpy as jnp
from jax.experimental import pallas as pl

_BLOCK_ROWS = 2048


def _pad_body(x_ref, out_ref, mask_ref):
    out_ref[...] = x_ref[...]

    @pl.when(pl.program_id(0) == 0)
    def _():
        mask_ref[...] = jnp.ones(mask_ref.shape, dtype=jnp.bool_)


def kernel(bags):
    b, s, d = bags.shape
    n = b * s
    flat = bags.reshape(n, d)
    rows = min(_BLOCK_ROWS, n)
    padded, mask = pl.pallas_call(
        _pad_body,
        grid=(pl.cdiv(n, rows),),
        in_specs=[pl.BlockSpec((rows, d), lambda i: (i, 0))],
        out_specs=(
            pl.BlockSpec((rows, d), lambda i: (i, 0)),
            pl.BlockSpec((b, s), lambda i: (0, 0)),
        ),
        out_shape=(
            jax.ShapeDtypeStruct((n, d), bags.dtype),
            jax.ShapeDtypeStruct((b, s), jnp.bool_),
        ),
    )(flat)
    return (padded.reshape(b, s, d), mask)
